# baseline (speedup 1.0000x reference)
import functools

import jax
import jax.numpy as jnp
from jax import lax
from jax.experimental import pallas as pl
from jax.experimental.pallas import tpu as pltpu
from jax.experimental.pallas import tpu_sc as plsc

C, H, W = 3, 512, 512
GH = GW = 8
TH = TW = 64
NB = 256
PIXELS = TH * TW
CLIP_VAL = 640.0
BAND = TH * W
LANES = 16
TROW = GW * NB
TYL = 5
N_BAND_CORE = C * TYL
QUARTER = BAND // 4
CHUNK_ROWS = 16
CHUNK_PX = CHUNK_ROWS * W

_mesh = plsc.VectorSubcoreMesh(
    core_axis_name="c", subcore_axis_name="s", num_cores=2, num_subcores=16
)
_params = pltpu.CompilerParams(needs_layout_passes=False)


def _floor_pos(x):
    return x.astype(jnp.int32).astype(jnp.float32)


def _lanes_f32():
    return lax.iota(jnp.int32, LANES).astype(jnp.float32)


def _lanes_i32():
    return lax.iota(jnp.int32, LANES)


@functools.partial(
    pl.kernel,
    out_type=jax.ShapeDtypeStruct((C * H * W,), jnp.float32),
    mesh=_mesh,
    compiler_params=_params,
    scratch_types=[
        pltpu.VMEM((BAND,), jnp.float32),
        pltpu.VMEM((2 * TROW,), jnp.float32),
        pltpu.VMEM((TROW,), jnp.float32),
        pltpu.VMEM((TROW,), jnp.float32),
        pltpu.VMEM_SHARED((N_BAND_CORE * TROW,), jnp.float32),
        pltpu.VMEM((C * 3 * TROW,), jnp.float32),
        pltpu.VMEM((CHUNK_PX,), jnp.float32),
        pltpu.VMEM((CHUNK_PX,), jnp.float32),
        pltpu.VMEM((CHUNK_PX,), jnp.float32),
        pltpu.VMEM((CHUNK_PX,), jnp.float32),
        pltpu.VMEM((W,), jnp.int32),
        pltpu.VMEM((W,), jnp.float32),
        pltpu.SemaphoreType.DMA,
        pltpu.SemaphoreType.DMA,
        pltpu.SemaphoreType.DMA,
        pltpu.SemaphoreType.DMA,
        pltpu.SemaphoreType.DMA,
        pltpu.SemaphoreType.DMA,
        pltpu.SemaphoreType.DMA,
        pltpu.SemaphoreType.DMA,
    ],
)
def _clahe_kernel(img_hbm, mask_hbm, out_hbm,
                  band_v, hsub_v, hist_v, lut_v, lut_sh, lsl_v,
                  in_v0, in_v1, mask_v, out_v, xo_v, wx_v,
                  s0, s1, s2, s3, sm, si0, si1, sl):
    core = lax.axis_index("c")
    sid = lax.axis_index("s")
    in_b = [in_v0, in_v1]
    isems = [si0, si1]

    g = sid + 16 * core

    mask_cp = pltpu.async_copy(
        mask_hbm.at[pl.ds(g * CHUNK_PX, CHUNK_PX)], mask_v, sm)
    img_cp0 = pltpu.async_copy(
        img_hbm.at[pl.ds(g * CHUNK_PX, CHUNK_PX)], in_v0, isems[0])

    @pl.when(sid < N_BAND_CORE)
    def _():
        cch = sid // TYL
        tyl = lax.rem(sid, TYL)
        ty = tyl + 3 * core
        band = cch * GH + ty
        sems = [s0, s1, s2, s3]
        copies = [
            pltpu.async_copy(
                img_hbm.at[pl.ds(band * BAND + q * QUARTER, QUARTER)],
                band_v.at[pl.ds(q * QUARTER, QUARTER)],
                sems[q],
            )
            for q in range(4)
        ]

        zeros = jnp.zeros((LANES,), jnp.float32)
        ones = jnp.ones((LANES,), jnp.float32)
        il = _lanes_i32()
        pix_patt = (il & 7) * TW + (il >> 3)
        hist_patt = (il >> 3) * TROW + (il & 7) * NB

        @plsc.parallel_loop(0, 2 * TROW // LANES, unroll=8)
        def zero_body(i):
            hsub_v[pl.ds(i * LANES, LANES)] = zeros

        for q in range(4):
            copies[q].wait()

            @plsc.parallel_loop(q * (QUARTER // LANES), (q + 1) * (QUARTER // LANES),
                                unroll=16)
            def hist_body(p):
                r = p >> 5
                j2 = p & 31
                base = r * W + j2 * 2
                px = plsc.load_gather(band_v, [pix_patt + jnp.full((LANES,), base, jnp.int32)])
                b = jnp.clip(px, 0.0, 255.0).astype(jnp.int32)
                plsc.addupdate_scatter(hsub_v, [hist_patt + b], ones)

        @plsc.parallel_loop(0, TROW // LANES, unroll=4)
        def merge_body(i):
            a = hsub_v[pl.ds(i * LANES, LANES)]
            b = hsub_v[pl.ds(TROW + i * LANES, LANES)]
            hist_v[pl.ds(i * LANES, LANES)] = jnp.minimum(a + b, CLIP_VAL)

        def tile_body(tx, carry):
            def sum_body(j, acc):
                return acc + hist_v[pl.ds(tx * NB + j * LANES, LANES)]

            sum_v = lax.fori_loop(0, NB // LANES, sum_body, zeros, unroll=4)
            total = jnp.full((LANES,), jnp.sum(sum_v))
            excess = jnp.float32(PIXELS) - total
            redist = _floor_pos(excess * jnp.float32(1.0 / NB))
            residual = excess - redist * jnp.float32(NB)

            def cum_body(j, carry_v):
                h = hist_v[pl.ds(tx * NB + j * LANES, LANES)]
                rng = _lanes_f32() + jnp.full((LANES,), j * LANES, jnp.float32)
                h2 = h + redist + jnp.where(rng < residual, 1.0, 0.0)
                cs = jnp.cumsum(h2) + carry_v
                lut_chunk = _floor_pos(
                    jnp.clip(cs * jnp.float32((NB - 1) / PIXELS), 0.0, jnp.float32(NB - 1))
                )
                lut_v[pl.ds(tx * NB + j * LANES, LANES)] = lut_chunk
                return carry_v + jnp.full((LANES,), jnp.sum(h2))

            lax.fori_loop(0, NB // LANES, cum_body, zeros, unroll=4)
            return carry

        lax.fori_loop(0, GW, tile_body, 0)
        pltpu.sync_copy(lut_v, lut_sh.at[pl.ds((cch * TYL + tyl) * TROW, TROW)])

    @plsc.parallel_loop(0, W // LANES, unroll=4)
    def xtab_body(xc):
        x = _lanes_f32() + jnp.full((LANES,), xc * LANES, jnp.float32)
        xx = (x + 0.5) * jnp.float32(1.0 / TW) - 0.5
        x0i = (xx + 8.0).astype(jnp.int32) - 8
        wx = xx - x0i.astype(jnp.float32)
        x0c = jnp.clip(x0i, 0, GW - 1)
        x1c = jnp.clip(x0i + 1, 0, GW - 1)
        xo_v[pl.ds(xc * LANES, LANES)] = x0c * NB | (x1c * NB) << 16
        wx_v[pl.ds(xc * LANES, LANES)] = wx

    plsc.subcore_barrier()

    num = 32 * g - 63
    a0 = jnp.where(num < 0, -1, num // 128)
    s_start = jnp.clip(a0, 3 * core, 3 * core + 2)
    sloc = s_start - 3 * core

    lut_cps = [
        pltpu.async_copy(
            lut_sh.at[pl.ds((k * TYL + sloc) * TROW, 3 * TROW)],
            lsl_v.at[pl.ds(k * 3 * TROW, 3 * TROW)],
            sl,
        )
        for k in range(C)
    ]
    for cp in lut_cps:
        cp.wait()
    mask_cp.wait()

    for k in range(C):
        buf = k % 2
        cid = k * 32 + g
        if k == 0:
            img_cp0.wait()
        if k + 1 < C:
            ncp = pltpu.async_copy(
                img_hbm.at[pl.ds((cid + 32) * CHUNK_PX, CHUNK_PX)],
                in_b[1 - buf], isems[1 - buf])
        lim_v = in_b[buf]

        def row_body(r, carry_r):
            y = g * CHUNK_ROWS + r
            y_f = jnp.full((LANES,), y, jnp.int32).astype(jnp.float32)
            yy = (y_f + 0.5) * jnp.float32(1.0 / TH) - 0.5
            y0i = (yy + 8.0).astype(jnp.int32) - 8
            wy = yy - y0i.astype(jnp.float32)
            wyc = 1.0 - wy
            ks = jnp.full((LANES,), 3 * k - s_start, jnp.int32)
            base0 = (jnp.clip(y0i, 0, GH - 1) + ks) * TROW
            base1 = (jnp.clip(y0i + 1, 0, GH - 1) + ks) * TROW

            @plsc.parallel_loop(0, W // LANES, unroll=16)
            def x_body(xc):
                off = r * W + xc * LANES
                px = lim_v[pl.ds(off, LANES)]
                m = mask_v[pl.ds(off, LANES)]
                b = jnp.clip(px, 0.0, 255.0).astype(jnp.int32)
                xo = xo_v[pl.ds(xc * LANES, LANES)]
                wx = wx_v[pl.ds(xc * LANES, LANES)]
                a0x = b + (xo & 65535)
                a1x = b + (xo >> 16)
                v00 = plsc.load_gather(lsl_v, [base0 + a0x])
                v01 = plsc.load_gather(lsl_v, [base0 + a1x])
                v10 = plsc.load_gather(lsl_v, [base1 + a0x])
                v11 = plsc.load_gather(lsl_v, [base1 + a1x])
                wxc = 1.0 - wx
                interp = wyc * (wxc * v00 + wx * v01) + wy * (wxc * v10 + wx * v11)
                eq = interp / jnp.float32(255.0)
                res = _floor_pos(jnp.clip(eq * jnp.float32(255.0), 0.0, 255.0)) * m
                out_v[pl.ds(off, LANES)] = res

            return carry_r

        lax.fori_loop(0, CHUNK_ROWS, row_body, 0)
        if k + 1 < C:
            ncp.wait()
        pltpu.sync_copy(out_v, out_hbm.at[pl.ds(cid * CHUNK_PX, CHUNK_PX)])


def kernel(image, label, keypoints, mask, probe):
    img_flat = image.reshape(-1)
    mask_flat = mask.reshape(-1)
    out_flat = _clahe_kernel(img_flat, mask_flat)
    new_image = out_flat.reshape(C, H, W)
    return (new_image, label, keypoints, mask, probe)


# speedup vs baseline: 1.0517x; 1.0517x over previous
import functools

import jax
import jax.numpy as jnp
from jax import lax
from jax.experimental import pallas as pl
from jax.experimental.pallas import tpu as pltpu
from jax.experimental.pallas import tpu_sc as plsc

C, H, W = 3, 512, 512
GH = GW = 8
TH = TW = 64
NB = 256
PIXELS = TH * TW
CLIP_VAL = 640.0
BAND = TH * W
LANES = 16
TROW = GW * NB
TYL = 5
N_BAND_CORE = C * TYL
QUARTER = BAND // 4
CHUNK_ROWS = 16
CHUNK_PX = CHUNK_ROWS * W

_mesh = plsc.VectorSubcoreMesh(
    core_axis_name="c", subcore_axis_name="s", num_cores=2, num_subcores=16
)
_params = pltpu.CompilerParams(needs_layout_passes=False)


def _floor_pos(x):
    return x.astype(jnp.int32).astype(jnp.float32)


def _lanes_f32():
    return lax.iota(jnp.int32, LANES).astype(jnp.float32)


def _lanes_i32():
    return lax.iota(jnp.int32, LANES)


@functools.partial(
    pl.kernel,
    out_type=jax.ShapeDtypeStruct((C * H * W,), jnp.float32),
    mesh=_mesh,
    compiler_params=_params,
    scratch_types=[
        pltpu.VMEM((BAND,), jnp.float32),
        pltpu.VMEM((2 * TROW,), jnp.float32),
        pltpu.VMEM((TROW,), jnp.float32),
        pltpu.VMEM((TROW,), jnp.float32),
        pltpu.VMEM_SHARED((N_BAND_CORE * TROW,), jnp.float32),
        pltpu.VMEM((C * 3 * TROW,), jnp.float32),
        pltpu.VMEM((CHUNK_PX,), jnp.float32),
        pltpu.VMEM((CHUNK_PX,), jnp.float32),
        pltpu.VMEM((CHUNK_PX,), jnp.float32),
        pltpu.VMEM((CHUNK_PX,), jnp.float32),
        pltpu.VMEM((CHUNK_PX,), jnp.float32),
        pltpu.VMEM((W,), jnp.int32),
        pltpu.VMEM((W,), jnp.float32),
        pltpu.SemaphoreType.DMA,
        pltpu.SemaphoreType.DMA,
        pltpu.SemaphoreType.DMA,
        pltpu.SemaphoreType.DMA,
        pltpu.SemaphoreType.DMA,
        pltpu.SemaphoreType.DMA,
        pltpu.SemaphoreType.DMA,
        pltpu.SemaphoreType.DMA,
        pltpu.SemaphoreType.DMA,
        pltpu.SemaphoreType.DMA,
    ],
)
def _clahe_kernel(img_hbm, mask_hbm, out_hbm,
                  band_v, hsub_v, hist_v, lut_v, lut_sh, lsl_v,
                  in_v0, in_v1, mask_v, out_v0, out_v1, xo_v, wx_v,
                  s0, s1, s2, s3, sm, si0, si1, sl, so0, so1):
    core = lax.axis_index("c")
    sid = lax.axis_index("s")
    in_b = [in_v0, in_v1]
    isems = [si0, si1]
    out_b = [out_v0, out_v1]
    osems = [so0, so1]

    g = sid + 16 * core

    mask_cp = pltpu.async_copy(
        mask_hbm.at[pl.ds(g * CHUNK_PX, CHUNK_PX)], mask_v, sm)
    img_cp0 = pltpu.async_copy(
        img_hbm.at[pl.ds(g * CHUNK_PX, CHUNK_PX)], in_v0, isems[0])

    @pl.when(sid < N_BAND_CORE)
    def _():
        cch = sid // TYL
        tyl = lax.rem(sid, TYL)
        ty = tyl + 3 * core
        band = cch * GH + ty
        sems = [s0, s1, s2, s3]
        copies = [
            pltpu.async_copy(
                img_hbm.at[pl.ds(band * BAND + q * QUARTER, QUARTER)],
                band_v.at[pl.ds(q * QUARTER, QUARTER)],
                sems[q],
            )
            for q in range(4)
        ]

        zeros = jnp.zeros((LANES,), jnp.float32)
        ones = jnp.ones((LANES,), jnp.float32)
        il = _lanes_i32()
        pix_patt = (il & 7) * TW + (il >> 3)
        hist_patt = (il >> 3) * TROW + (il & 7) * NB

        @plsc.parallel_loop(0, 2 * TROW // LANES, unroll=8)
        def zero_body(i):
            hsub_v[pl.ds(i * LANES, LANES)] = zeros

        for q in range(4):
            copies[q].wait()

            @plsc.parallel_loop(q * (QUARTER // LANES), (q + 1) * (QUARTER // LANES),
                                unroll=8)
            def hist_body(p):
                r = p >> 5
                j2 = p & 31
                base = r * W + j2 * 2
                px = plsc.load_gather(band_v, [pix_patt + jnp.full((LANES,), base, jnp.int32)])
                b = jnp.clip(px, 0.0, 255.0).astype(jnp.int32)
                plsc.addupdate_scatter(hsub_v, [hist_patt + b], ones)

        @plsc.parallel_loop(0, TROW // LANES, unroll=4)
        def merge_body(i):
            a = hsub_v[pl.ds(i * LANES, LANES)]
            b = hsub_v[pl.ds(TROW + i * LANES, LANES)]
            hist_v[pl.ds(i * LANES, LANES)] = jnp.minimum(a + b, CLIP_VAL)

        def tile_body(tx, carry):
            def sum_body(j, acc):
                return acc + hist_v[pl.ds(tx * NB + j * LANES, LANES)]

            sum_v = lax.fori_loop(0, NB // LANES, sum_body, zeros, unroll=4)
            total = jnp.full((LANES,), jnp.sum(sum_v))
            excess = jnp.float32(PIXELS) - total
            redist = _floor_pos(excess * jnp.float32(1.0 / NB))
            residual = excess - redist * jnp.float32(NB)

            def cum_body(j, carry_v):
                h = hist_v[pl.ds(tx * NB + j * LANES, LANES)]
                rng = _lanes_f32() + jnp.full((LANES,), j * LANES, jnp.float32)
                h2 = h + redist + jnp.where(rng < residual, 1.0, 0.0)
                cs = jnp.cumsum(h2) + carry_v
                lut_chunk = _floor_pos(
                    jnp.clip(cs * jnp.float32((NB - 1) / PIXELS), 0.0, jnp.float32(NB - 1))
                )
                lut_v[pl.ds(tx * NB + j * LANES, LANES)] = lut_chunk
                return carry_v + jnp.full((LANES,), jnp.sum(h2))

            lax.fori_loop(0, NB // LANES, cum_body, zeros, unroll=4)
            return carry

        lax.fori_loop(0, GW, tile_body, 0)
        pltpu.sync_copy(lut_v, lut_sh.at[pl.ds((cch * TYL + tyl) * TROW, TROW)])

    @plsc.parallel_loop(0, W // LANES, unroll=4)
    def xtab_body(xc):
        x = _lanes_f32() + jnp.full((LANES,), xc * LANES, jnp.float32)
        xx = (x + 0.5) * jnp.float32(1.0 / TW) - 0.5
        x0i = (xx + 8.0).astype(jnp.int32) - 8
        wx = xx - x0i.astype(jnp.float32)
        x0c = jnp.clip(x0i, 0, GW - 1)
        x1c = jnp.clip(x0i + 1, 0, GW - 1)
        xo_v[pl.ds(xc * LANES, LANES)] = x0c * NB | (x1c * NB) << 16
        wx_v[pl.ds(xc * LANES, LANES)] = wx

    plsc.subcore_barrier()

    num = 32 * g - 63
    a0 = jnp.where(num < 0, -1, num // 128)
    s_start = jnp.clip(a0, 3 * core, 3 * core + 2)
    sloc = s_start - 3 * core

    lut_cps = [
        pltpu.async_copy(
            lut_sh.at[pl.ds((k * TYL + sloc) * TROW, 3 * TROW)],
            lsl_v.at[pl.ds(k * 3 * TROW, 3 * TROW)],
            sl,
        )
        for k in range(C)
    ]
    for cp in lut_cps:
        cp.wait()
    mask_cp.wait()

    out_cps = [None, None, None]
    for k in range(C):
        buf = k % 2
        cid = k * 32 + g
        if k == 0:
            img_cp0.wait()
        if k == 2:
            out_cps[0].wait()
        out_v = out_b[buf]
        if k + 1 < C:
            ncp = pltpu.async_copy(
                img_hbm.at[pl.ds((cid + 32) * CHUNK_PX, CHUNK_PX)],
                in_b[1 - buf], isems[1 - buf])
        lim_v = in_b[buf]

        def row_body(r, carry_r):
            y = g * CHUNK_ROWS + r
            y_f = jnp.full((LANES,), y, jnp.int32).astype(jnp.float32)
            yy = (y_f + 0.5) * jnp.float32(1.0 / TH) - 0.5
            y0i = (yy + 8.0).astype(jnp.int32) - 8
            wy = yy - y0i.astype(jnp.float32)
            wyc = 1.0 - wy
            ks = jnp.full((LANES,), 3 * k - s_start, jnp.int32)
            base0 = (jnp.clip(y0i, 0, GH - 1) + ks) * TROW
            base1 = (jnp.clip(y0i + 1, 0, GH - 1) + ks) * TROW

            @plsc.parallel_loop(0, W // LANES, unroll=8)
            def x_body(xc):
                off = r * W + xc * LANES
                px = lim_v[pl.ds(off, LANES)]
                m = mask_v[pl.ds(off, LANES)]
                b = jnp.clip(px, 0.0, 255.0).astype(jnp.int32)
                xo = xo_v[pl.ds(xc * LANES, LANES)]
                wx = wx_v[pl.ds(xc * LANES, LANES)]
                a0x = b + (xo & 65535)
                a1x = b + (xo >> 16)
                v00 = plsc.load_gather(lsl_v, [base0 + a0x])
                v01 = plsc.load_gather(lsl_v, [base0 + a1x])
                v10 = plsc.load_gather(lsl_v, [base1 + a0x])
                v11 = plsc.load_gather(lsl_v, [base1 + a1x])
                wxc = 1.0 - wx
                interp = wyc * (wxc * v00 + wx * v01) + wy * (wxc * v10 + wx * v11)
                eq = interp / jnp.float32(255.0)
                res = _floor_pos(jnp.clip(eq * jnp.float32(255.0), 0.0, 255.0)) * m
                out_v[pl.ds(off, LANES)] = res

            return carry_r

        lax.fori_loop(0, CHUNK_ROWS, row_body, 0)
        if k + 1 < C:
            ncp.wait()
        out_cps[k] = pltpu.async_copy(
            out_v, out_hbm.at[pl.ds(cid * CHUNK_PX, CHUNK_PX)], osems[buf])

    out_cps[1].wait()
    out_cps[2].wait()


def kernel(image, label, keypoints, mask, probe):
    img_flat = image.reshape(-1)
    mask_flat = mask.reshape(-1)
    out_flat = _clahe_kernel(img_flat, mask_flat)
    new_image = out_flat.reshape(C, H, W)
    return (new_image, label, keypoints, mask, probe)


# speedup vs baseline: 1.0532x; 1.0014x over previous
import functools

import jax
import jax.numpy as jnp
from jax import lax
from jax.experimental import pallas as pl
from jax.experimental.pallas import tpu as pltpu
from jax.experimental.pallas import tpu_sc as plsc

C, H, W = 3, 512, 512
GH = GW = 8
TH = TW = 64
NB = 256
PIXELS = TH * TW
CLIP_VAL = 640.0
BAND = TH * W
LANES = 16
TROW = GW * NB
TYL = 5
N_BAND_CORE = C * TYL
QUARTER = BAND // 8
CHUNK_ROWS = 16
CHUNK_PX = CHUNK_ROWS * W

_mesh = plsc.VectorSubcoreMesh(
    core_axis_name="c", subcore_axis_name="s", num_cores=2, num_subcores=16
)
_params = pltpu.CompilerParams(needs_layout_passes=False)


def _floor_pos(x):
    return x.astype(jnp.int32).astype(jnp.float32)


def _lanes_f32():
    return lax.iota(jnp.int32, LANES).astype(jnp.float32)


def _lanes_i32():
    return lax.iota(jnp.int32, LANES)


@functools.partial(
    pl.kernel,
    out_type=jax.ShapeDtypeStruct((C * H * W,), jnp.float32),
    mesh=_mesh,
    compiler_params=_params,
    scratch_types=[
        pltpu.VMEM((BAND,), jnp.float32),
        pltpu.VMEM((2 * TROW,), jnp.float32),
        pltpu.VMEM((TROW,), jnp.float32),
        pltpu.VMEM((TROW,), jnp.float32),
        pltpu.VMEM_SHARED((N_BAND_CORE * TROW,), jnp.float32),
        pltpu.VMEM((C * 3 * TROW,), jnp.float32),
        pltpu.VMEM((CHUNK_PX,), jnp.float32),
        pltpu.VMEM((CHUNK_PX,), jnp.float32),
        pltpu.VMEM((CHUNK_PX,), jnp.float32),
        pltpu.VMEM((CHUNK_PX,), jnp.float32),
        pltpu.VMEM((CHUNK_PX,), jnp.float32),
        pltpu.VMEM((W,), jnp.int32),
        pltpu.VMEM((W,), jnp.float32),
        pltpu.SemaphoreType.DMA,
        pltpu.SemaphoreType.DMA,
        pltpu.SemaphoreType.DMA,
        pltpu.SemaphoreType.DMA,
        pltpu.SemaphoreType.DMA,
        pltpu.SemaphoreType.DMA,
        pltpu.SemaphoreType.DMA,
        pltpu.SemaphoreType.DMA,
        pltpu.SemaphoreType.DMA,
        pltpu.SemaphoreType.DMA,
        pltpu.SemaphoreType.DMA,
        pltpu.SemaphoreType.DMA,
        pltpu.SemaphoreType.DMA,
        pltpu.SemaphoreType.DMA,
    ],
)
def _clahe_kernel(img_hbm, mask_hbm, out_hbm,
                  band_v, hsub_v, hist_v, lut_v, lut_sh, lsl_v,
                  in_v0, in_v1, mask_v, out_v0, out_v1, xo_v, wx_v,
                  s0, s1, s2, s3, s4, s5, s6, s7, sm, si0, si1, sl, so0, so1):
    core = lax.axis_index("c")
    sid = lax.axis_index("s")
    in_b = [in_v0, in_v1]
    isems = [si0, si1]
    out_b = [out_v0, out_v1]
    osems = [so0, so1]

    g = sid + 16 * core

    mask_cp = pltpu.async_copy(
        mask_hbm.at[pl.ds(g * CHUNK_PX, CHUNK_PX)], mask_v, sm)
    img_cp0 = pltpu.async_copy(
        img_hbm.at[pl.ds(g * CHUNK_PX, CHUNK_PX)], in_v0, isems[0])

    @pl.when(sid < N_BAND_CORE)
    def _():
        cch = sid // TYL
        tyl = lax.rem(sid, TYL)
        ty = tyl + 3 * core
        band = cch * GH + ty
        sems = [s0, s1, s2, s3, s4, s5, s6, s7]
        copies = [
            pltpu.async_copy(
                img_hbm.at[pl.ds(band * BAND + q * QUARTER, QUARTER)],
                band_v.at[pl.ds(q * QUARTER, QUARTER)],
                sems[q],
            )
            for q in range(8)
        ]

        zeros = jnp.zeros((LANES,), jnp.float32)
        ones = jnp.ones((LANES,), jnp.float32)
        il = _lanes_i32()
        pix_patt = (il & 7) * TW + (il >> 3)
        hist_patt = (il >> 3) * TROW + (il & 7) * NB

        @plsc.parallel_loop(0, 2 * TROW // LANES, unroll=8)
        def zero_body(i):
            hsub_v[pl.ds(i * LANES, LANES)] = zeros

        for q in range(8):
            copies[q].wait()

            @plsc.parallel_loop(q * (QUARTER // LANES), (q + 1) * (QUARTER // LANES),
                                unroll=8)
            def hist_body(p):
                r = p >> 5
                j2 = p & 31
                base = r * W + j2 * 2
                px = plsc.load_gather(band_v, [pix_patt + jnp.full((LANES,), base, jnp.int32)])
                b = jnp.clip(px, 0.0, 255.0).astype(jnp.int32)
                plsc.addupdate_scatter(hsub_v, [hist_patt + b], ones)

        @plsc.parallel_loop(0, TROW // LANES, unroll=4)
        def merge_body(i):
            a = hsub_v[pl.ds(i * LANES, LANES)]
            b = hsub_v[pl.ds(TROW + i * LANES, LANES)]
            hist_v[pl.ds(i * LANES, LANES)] = jnp.minimum(a + b, CLIP_VAL)

        def tile_body(tx, carry):
            def sum_body(j, acc):
                return acc + hist_v[pl.ds(tx * NB + j * LANES, LANES)]

            sum_v = lax.fori_loop(0, NB // LANES, sum_body, zeros, unroll=4)
            total = jnp.full((LANES,), jnp.sum(sum_v))
            excess = jnp.float32(PIXELS) - total
            redist = _floor_pos(excess * jnp.float32(1.0 / NB))
            residual = excess - redist * jnp.float32(NB)

            def cum_body(j, carry_v):
                h = hist_v[pl.ds(tx * NB + j * LANES, LANES)]
                rng = _lanes_f32() + jnp.full((LANES,), j * LANES, jnp.float32)
                h2 = h + redist + jnp.where(rng < residual, 1.0, 0.0)
                cs = jnp.cumsum(h2) + carry_v
                lut_chunk = _floor_pos(
                    jnp.clip(cs * jnp.float32((NB - 1) / PIXELS), 0.0, jnp.float32(NB - 1))
                )
                lut_v[pl.ds(tx * NB + j * LANES, LANES)] = lut_chunk
                return carry_v + jnp.full((LANES,), jnp.sum(h2))

            lax.fori_loop(0, NB // LANES, cum_body, zeros, unroll=4)
            return carry

        lax.fori_loop(0, GW, tile_body, 0)
        pltpu.sync_copy(lut_v, lut_sh.at[pl.ds((cch * TYL + tyl) * TROW, TROW)])

    @plsc.parallel_loop(0, W // LANES, unroll=4)
    def xtab_body(xc):
        x = _lanes_f32() + jnp.full((LANES,), xc * LANES, jnp.float32)
        xx = (x + 0.5) * jnp.float32(1.0 / TW) - 0.5
        x0i = (xx + 8.0).astype(jnp.int32) - 8
        wx = xx - x0i.astype(jnp.float32)
        x0c = jnp.clip(x0i, 0, GW - 1)
        x1c = jnp.clip(x0i + 1, 0, GW - 1)
        xo_v[pl.ds(xc * LANES, LANES)] = x0c * NB | (x1c * NB) << 16
        wx_v[pl.ds(xc * LANES, LANES)] = wx

    plsc.subcore_barrier()

    num = 32 * g - 63
    a0 = jnp.where(num < 0, -1, num // 128)
    s_start = jnp.clip(a0, 3 * core, 3 * core + 2)
    sloc = s_start - 3 * core

    lut_cps = [
        pltpu.async_copy(
            lut_sh.at[pl.ds((k * TYL + sloc) * TROW, 3 * TROW)],
            lsl_v.at[pl.ds(k * 3 * TROW, 3 * TROW)],
            sl,
        )
        for k in range(C)
    ]
    for cp in lut_cps:
        cp.wait()
    mask_cp.wait()

    out_cps = [None, None, None]
    for k in range(C):
        buf = k % 2
        cid = k * 32 + g
        if k == 0:
            img_cp0.wait()
        if k == 2:
            out_cps[0].wait()
        out_v = out_b[buf]
        if k + 1 < C:
            ncp = pltpu.async_copy(
                img_hbm.at[pl.ds((cid + 32) * CHUNK_PX, CHUNK_PX)],
                in_b[1 - buf], isems[1 - buf])
        lim_v = in_b[buf]

        def row_body(r, carry_r):
            y = g * CHUNK_ROWS + r
            y_f = jnp.full((LANES,), y, jnp.int32).astype(jnp.float32)
            yy = (y_f + 0.5) * jnp.float32(1.0 / TH) - 0.5
            y0i = (yy + 8.0).astype(jnp.int32) - 8
            wy = yy - y0i.astype(jnp.float32)
            wyc = 1.0 - wy
            ks = jnp.full((LANES,), 3 * k - s_start, jnp.int32)
            base0 = (jnp.clip(y0i, 0, GH - 1) + ks) * TROW
            base1 = (jnp.clip(y0i + 1, 0, GH - 1) + ks) * TROW

            @plsc.parallel_loop(0, W // LANES, unroll=8)
            def x_body(xc):
                off = r * W + xc * LANES
                px = lim_v[pl.ds(off, LANES)]
                m = mask_v[pl.ds(off, LANES)]
                b = jnp.clip(px, 0.0, 255.0).astype(jnp.int32)
                xo = xo_v[pl.ds(xc * LANES, LANES)]
                wx = wx_v[pl.ds(xc * LANES, LANES)]
                a0x = b + (xo & 65535)
                a1x = b + (xo >> 16)
                v00 = plsc.load_gather(lsl_v, [base0 + a0x])
                v01 = plsc.load_gather(lsl_v, [base0 + a1x])
                v10 = plsc.load_gather(lsl_v, [base1 + a0x])
                v11 = plsc.load_gather(lsl_v, [base1 + a1x])
                wxc = 1.0 - wx
                interp = wyc * (wxc * v00 + wx * v01) + wy * (wxc * v10 + wx * v11)
                eq = interp / jnp.float32(255.0)
                res = _floor_pos(jnp.clip(eq * jnp.float32(255.0), 0.0, 255.0)) * m
                out_v[pl.ds(off, LANES)] = res

            return carry_r

        lax.fori_loop(0, CHUNK_ROWS, row_body, 0)
        if k + 1 < C:
            ncp.wait()
        out_cps[k] = pltpu.async_copy(
            out_v, out_hbm.at[pl.ds(cid * CHUNK_PX, CHUNK_PX)], osems[buf])

    out_cps[1].wait()
    out_cps[2].wait()


def kernel(image, label, keypoints, mask, probe):
    img_flat = image.reshape(-1)
    mask_flat = mask.reshape(-1)
    out_flat = _clahe_kernel(img_flat, mask_flat)
    new_image = out_flat.reshape(C, H, W)
    return (new_image, label, keypoints, mask, probe)
